# baseline (speedup 1.0000x reference)
import functools
import math

import jax
import jax.numpy as jnp
import numpy as np
from jax import lax
from jax.experimental import pallas as pl
from jax.experimental.pallas import tpu as pltpu
from jax.experimental.pallas import tpu_sc as plsc

T = 524288
L = 16
F = 2
N_MIN = 16
N_MAX = 2048
NUM_POINTS = 262144

NC = 2
NS = 16
NW = NC * NS
LANES = 16

PW = NUM_POINTS // NW
B = 512
NBLK = PW // B
BG = B // LANES
ROWS = B * 8
TCHUNK = T // NS
SG = 24
SROWS = SG * 8 * LANES
HROWS = ROWS - SROWS

_GROWTH = math.exp((math.log(N_MAX) - math.log(N_MIN)) / (L - 1))
SCALES = [float(math.floor(N_MIN * (_GROWTH ** l))) for l in range(L)]
P2 = -1640531535
P3 = 805459861
MASK = T - 1


def _body(xs_hbm, ys_hbm, zs_hbm, t0_hbm, t1_hbm, sc_hbm, out_hbm,
          xv, yv, zv, fx0, fy0, fz0, fx1, fy1, fz1,
          idx0, idx1, ixg0, ixg1, g0a, g1a, g0b, g1b, ova, ovvb, scv,
          tsh0, tsh1,
          sa0, sa1, sb0, sb1, ha0, ha1, hb0, hb1, so0, so1):
    ovb = (ova, ovvb)
    osem = (so0, so1)
    idxb = (idx0, idx1)
    idxgb = (ixg0, ixg1)
    gb = ((g0a, g1a), (g0b, g1b))
    semb = ((sa0, sa1), (sb0, sb1))
    hsemb = ((ha0, ha1), (hb0, hb1))
    fracb = ((fx0, fy0, fz0), (fx1, fy1, fz1))

    wid = lax.axis_index("s") * NC + lax.axis_index("c")
    sid = lax.axis_index("s")
    wbase = wid * PW
    pltpu.sync_copy(xs_hbm.at[pl.ds(wbase, PW)], xv)
    pltpu.sync_copy(ys_hbm.at[pl.ds(wbase, PW)], yv)
    pltpu.sync_copy(zs_hbm.at[pl.ds(wbase, PW)], zv)
    pltpu.sync_copy(sc_hbm, scv)

    def make_idx_loop(boff, scale, lvl, p):
        idxv = idxb[p]
        idxg = idxgb[p]
        fxv, fyv, fzv = fracb[p]
        lvl_t = lvl * T

        def hash_groups(g, store):
            sl = pl.ds(g * LANES, LANES)
            xsl = pl.ds(boff + g * LANES, LANES)
            px = xv[xsl] * scale
            py = yv[xsl] * scale
            pz = zv[xsl] * scale
            ix = px.astype(jnp.int32)
            iy = py.astype(jnp.int32)
            iz = pz.astype(jnp.int32)
            fxv[sl] = px - ix.astype(jnp.float32)
            fyv[sl] = py - iy.astype(jnp.float32)
            fzv[sl] = pz - iz.astype(jnp.float32)
            yp = iy * P2
            zp = iz * P3
            hy1 = yp + P2
            hz1 = zp + P3
            a0 = ix & MASK
            a1 = (ix + 1) & MASK
            b00 = (yp ^ zp) & MASK
            b01 = (yp ^ hz1) & MASK
            b10 = (hy1 ^ zp) & MASK
            b11 = (hy1 ^ hz1) & MASK
            store(a0, a1, (b00, b01, b10, b11))

        @plsc.parallel_loop(0, SG, unroll=2)
        def idx_body(g):
            def store(a0, a1, bs):
                rb = g * (8 * LANES)
                for i, a in enumerate((a0, a1)):
                    for j, b in enumerate(bs):
                        idxv[pl.ds(rb + (4 * i + j) * LANES, LANES)] = a ^ b
            hash_groups(g, store)

        @plsc.parallel_loop(SG, BG, unroll=2)
        def idxg_body(g):
            def store(a0, a1, bs):
                rb = (g - SG) * (8 * LANES)
                a0g = a0 ^ lvl_t
                a1g = a1 ^ lvl_t
                for i, a in enumerate((a0g, a1g)):
                    for j, b in enumerate(bs):
                        idxg[pl.ds(rb + (4 * i + j) * LANES, LANES)] = a ^ b
            hash_groups(g, store)

    def fire(p):
        pltpu.async_copy(tsh0.at[idxb[p]],
                         gb[p][0].at[pl.ds(0, SROWS)], semb[p][0])
        pltpu.async_copy(tsh1.at[idxb[p]],
                         gb[p][1].at[pl.ds(0, SROWS)], semb[p][1])
        pltpu.async_copy(t0_hbm.at[idxgb[p]],
                         gb[p][0].at[pl.ds(SROWS, HROWS)], hsemb[p][0])
        pltpu.async_copy(t1_hbm.at[idxgb[p]],
                         gb[p][1].at[pl.ds(SROWS, HROWS)], hsemb[p][1])

    def gwait(p):
        pltpu.make_async_copy(tsh0.at[idxb[p]],
                              gb[p][0].at[pl.ds(0, SROWS)], semb[p][0]).wait()
        pltpu.make_async_copy(tsh1.at[idxb[p]],
                              gb[p][1].at[pl.ds(0, SROWS)], semb[p][1]).wait()
        pltpu.make_async_copy(t0_hbm.at[idxgb[p]],
                              gb[p][0].at[pl.ds(SROWS, HROWS)],
                              hsemb[p][0]).wait()
        pltpu.make_async_copy(t1_hbm.at[idxgb[p]],
                              gb[p][1].at[pl.ds(SROWS, HROWS)],
                              hsemb[p][1]).wait()

    def make_comb_loop(p):
        ovl = ovb[p]
        g0v, g1v = gb[p]
        fxv, fyv, fzv = fracb[p]

        @plsc.parallel_loop(0, BG, unroll=2)
        def comb_body(g):
            sl = pl.ds(g * LANES, LANES)
            fx = fxv[sl]
            fy = fyv[sl]
            fz = fzv[sl]
            gx = 1.0 - fx
            gy = 1.0 - fy
            gz = 1.0 - fz
            w00 = gx * gy
            w01 = gx * fy
            w10 = fx * gy
            w11 = fx * fy
            ws = (w00 * gz, w00 * fz, w01 * gz, w01 * fz,
                  w10 * gz, w10 * fz, w11 * gz, w11 * fz)
            rb = g * (8 * LANES)
            acc0 = None
            acc1 = None
            for c in range(8):
                f0 = g0v[pl.ds(rb + c * LANES, LANES)]
                f1 = g1v[pl.ds(rb + c * LANES, LANES)]
                if c == 0:
                    acc0 = ws[c] * f0
                    acc1 = ws[c] * f1
                else:
                    acc0 = acc0 + ws[c] * f0
                    acc1 = acc1 + ws[c] * f1
            ovl[0, sl] = acc0
            ovl[1, sl] = acc1

    def ofire(row0, blk, p):
        obase = row0 + wbase + blk * B
        pltpu.async_copy(ovb[p].at[0], out_hbm.at[pl.ds(obase, B)], osem[p])
        pltpu.async_copy(
            ovb[p].at[1], out_hbm.at[pl.ds(obase + NUM_POINTS, B)], osem[p])

    def owait(p):
        pltpu.make_async_copy(
            ovb[p].at[0], out_hbm.at[pl.ds(0, B)], osem[p]).wait()
        pltpu.make_async_copy(
            ovb[p].at[1], out_hbm.at[pl.ds(0, B)], osem[p]).wait()

    def lvl_body(lvl, carry):
        cbase = lvl * T + sid * TCHUNK
        pltpu.sync_copy(t0_hbm.at[pl.ds(cbase, TCHUNK)],
                        tsh0.at[pl.ds(sid * TCHUNK, TCHUNK)])
        pltpu.sync_copy(t1_hbm.at[pl.ds(cbase, TCHUNK)],
                        tsh1.at[pl.ds(sid * TCHUNK, TCHUNK)])
        plsc.subcore_barrier()

        scale = scv[pl.ds(lvl * LANES, LANES)]
        row0 = 2 * lvl * NUM_POINTS

        make_idx_loop(0, scale, lvl, 0)
        fire(0)
        make_idx_loop(B, scale, lvl, 1)
        gwait(0)
        fire(1)
        make_comb_loop(0)
        ofire(row0, 0, 0)
        make_idx_loop(2 * B, scale, lvl, 0)
        gwait(1)
        fire(0)
        make_comb_loop(1)
        ofire(row0, 1, 1)

        def blk2_body(i, c0):
            b0 = 2 * i
            make_idx_loop((b0 + 1) * B, scale, lvl, 1)
            gwait(0)
            fire(1)
            owait(0)
            make_comb_loop(0)
            ofire(row0, b0, 0)
            make_idx_loop((b0 + 2) * B, scale, lvl, 0)
            gwait(1)
            fire(0)
            owait(1)
            make_comb_loop(1)
            ofire(row0, b0 + 1, 1)
            return c0

        lax.fori_loop(1, NBLK // 2 - 1, blk2_body, 0)

        make_idx_loop(15 * B, scale, lvl, 1)
        gwait(0)
        fire(1)
        owait(0)
        make_comb_loop(0)
        ofire(row0, 14, 0)
        gwait(1)
        owait(1)
        make_comb_loop(1)
        ofire(row0, 15, 1)
        owait(0)
        owait(1)
        plsc.subcore_barrier()
        return carry

    lax.fori_loop(0, L, lvl_body, 0)


@jax.jit
def _encode_sc(xs, ys, zs, t0, t1, scales):
    mesh = plsc.VectorSubcoreMesh(core_axis_name="c", subcore_axis_name="s")
    return pl.kernel(
        _body,
        out_type=jax.ShapeDtypeStruct((L * F * NUM_POINTS,), jnp.float32),
        mesh=mesh,
        scratch_types=[
            pltpu.VMEM((PW,), jnp.float32),
            pltpu.VMEM((PW,), jnp.float32),
            pltpu.VMEM((PW,), jnp.float32),
            pltpu.VMEM((B,), jnp.float32),
            pltpu.VMEM((B,), jnp.float32),
            pltpu.VMEM((B,), jnp.float32),
            pltpu.VMEM((B,), jnp.float32),
            pltpu.VMEM((B,), jnp.float32),
            pltpu.VMEM((B,), jnp.float32),
            pltpu.VMEM((SROWS,), jnp.int32),
            pltpu.VMEM((SROWS,), jnp.int32),
            pltpu.VMEM((HROWS,), jnp.int32),
            pltpu.VMEM((HROWS,), jnp.int32),
            pltpu.VMEM((ROWS,), jnp.float32),
            pltpu.VMEM((ROWS,), jnp.float32),
            pltpu.VMEM((ROWS,), jnp.float32),
            pltpu.VMEM((ROWS,), jnp.float32),
            pltpu.VMEM((F, B), jnp.float32),
            pltpu.VMEM((F, B), jnp.float32),
            pltpu.VMEM((L * LANES,), jnp.float32),
            pltpu.VMEM_SHARED((T,), jnp.float32),
            pltpu.VMEM_SHARED((T,), jnp.float32),
            pltpu.SemaphoreType.DMA,
            pltpu.SemaphoreType.DMA,
            pltpu.SemaphoreType.DMA,
            pltpu.SemaphoreType.DMA,
            pltpu.SemaphoreType.DMA,
            pltpu.SemaphoreType.DMA,
            pltpu.SemaphoreType.DMA,
            pltpu.SemaphoreType.DMA,
            pltpu.SemaphoreType.DMA,
            pltpu.SemaphoreType.DMA,
        ],
    )(xs, ys, zs, t0, t1, scales)


def kernel(x, hash_table):
    xs, ys, zs = x[:, 0], x[:, 1], x[:, 2]
    t0, t1 = hash_table[:, 0], hash_table[:, 1]
    scales = jnp.asarray(
        np.repeat(np.array(SCALES, dtype=np.float32), LANES))
    out = _encode_sc(xs, ys, zs, t0, t1, scales)
    return out.reshape(L * F, NUM_POINTS).T


# speedup vs baseline: 1.0869x; 1.0869x over previous
import functools
import math

import jax
import jax.numpy as jnp
import numpy as np
from jax import lax
from jax.experimental import pallas as pl
from jax.experimental.pallas import tpu as pltpu
from jax.experimental.pallas import tpu_sc as plsc

T = 524288
L = 16
F = 2
N_MIN = 16
N_MAX = 2048
NUM_POINTS = 262144

NC = 2
NS = 16
NW = NC * NS
LANES = 16

PW = NUM_POINTS // NW
B = 512
NBLK = PW // B
BG = B // LANES
ROWS = B * 8
TCHUNK = T // NS
SG = 24
SROWS = SG * 8 * LANES
HROWS = ROWS - SROWS

_GROWTH = math.exp((math.log(N_MAX) - math.log(N_MIN)) / (L - 1))
SCALES = [float(math.floor(N_MIN * (_GROWTH ** l))) for l in range(L)]
P2 = -1640531535
P3 = 805459861
MASK = T - 1


def _body(xs_hbm, ys_hbm, zs_hbm, t0_hbm, t1_hbm, sc_hbm, out_hbm,
          xv, yv, zv, fx0, fy0, fz0, fx1, fy1, fz1,
          idx0, idx1, g0a, g1a, g0b, g1b, ova, ovvb, scv,
          tsh0, tsh1,
          sa0, sa1, sb0, sb1, so0, so1):
    ovb = (ova, ovvb)
    osem = (so0, so1)
    idxb = (idx0, idx1)
    gb = ((g0a, g1a), (g0b, g1b))
    semb = ((sa0, sa1), (sb0, sb1))
    fracb = ((fx0, fy0, fz0), (fx1, fy1, fz1))

    wid = lax.axis_index("s") * NC + lax.axis_index("c")
    sid = lax.axis_index("s")
    wbase = wid * PW
    pltpu.sync_copy(xs_hbm.at[pl.ds(wbase, PW)], xv)
    pltpu.sync_copy(ys_hbm.at[pl.ds(wbase, PW)], yv)
    pltpu.sync_copy(zs_hbm.at[pl.ds(wbase, PW)], zv)
    pltpu.sync_copy(sc_hbm, scv)

    def make_idx_loop(boff, scale, lvl, p):
        idxv = idxb[p]
        fxv, fyv, fzv = fracb[p]

        def hash_groups(g, store):
            sl = pl.ds(g * LANES, LANES)
            xsl = pl.ds(boff + g * LANES, LANES)
            px = xv[xsl] * scale
            py = yv[xsl] * scale
            pz = zv[xsl] * scale
            ix = px.astype(jnp.int32)
            iy = py.astype(jnp.int32)
            iz = pz.astype(jnp.int32)
            fxv[sl] = px - ix.astype(jnp.float32)
            fyv[sl] = py - iy.astype(jnp.float32)
            fzv[sl] = pz - iz.astype(jnp.float32)
            yp = iy * P2
            zp = iz * P3
            hy1 = yp + P2
            hz1 = zp + P3
            a0 = ix & MASK
            a1 = (ix + 1) & MASK
            b00 = (yp ^ zp) & MASK
            b01 = (yp ^ hz1) & MASK
            b10 = (hy1 ^ zp) & MASK
            b11 = (hy1 ^ hz1) & MASK
            store(a0, a1, (b00, b01, b10, b11))

        @plsc.parallel_loop(0, BG, unroll=4)
        def idx_body(g):
            def store(a0, a1, bs):
                rb = g * (8 * LANES)
                for i, a in enumerate((a0, a1)):
                    for j, b in enumerate(bs):
                        idxv[pl.ds(rb + (4 * i + j) * LANES, LANES)] = a ^ b
            hash_groups(g, store)

    def fire(p):
        pltpu.async_copy(tsh0.at[idxb[p]], gb[p][0], semb[p][0])
        pltpu.async_copy(tsh1.at[idxb[p]], gb[p][1], semb[p][1])

    def gwait(p):
        pltpu.make_async_copy(tsh0.at[idxb[p]], gb[p][0], semb[p][0]).wait()
        pltpu.make_async_copy(tsh1.at[idxb[p]], gb[p][1], semb[p][1]).wait()

    def make_comb_loop(p):
        ovl = ovb[p]
        g0v, g1v = gb[p]
        fxv, fyv, fzv = fracb[p]

        @plsc.parallel_loop(0, BG, unroll=4)
        def comb_body(g):
            sl = pl.ds(g * LANES, LANES)
            fx = fxv[sl]
            fy = fyv[sl]
            fz = fzv[sl]
            gx = 1.0 - fx
            gy = 1.0 - fy
            gz = 1.0 - fz
            w00 = gx * gy
            w01 = gx * fy
            w10 = fx * gy
            w11 = fx * fy
            ws = (w00 * gz, w00 * fz, w01 * gz, w01 * fz,
                  w10 * gz, w10 * fz, w11 * gz, w11 * fz)
            rb = g * (8 * LANES)
            acc0 = None
            acc1 = None
            for c in range(8):
                f0 = g0v[pl.ds(rb + c * LANES, LANES)]
                f1 = g1v[pl.ds(rb + c * LANES, LANES)]
                if c == 0:
                    acc0 = ws[c] * f0
                    acc1 = ws[c] * f1
                else:
                    acc0 = acc0 + ws[c] * f0
                    acc1 = acc1 + ws[c] * f1
            ovl[0, sl] = acc0
            ovl[1, sl] = acc1

    def ofire(row0, blk, p):
        obase = row0 + wbase + blk * B
        pltpu.async_copy(ovb[p].at[0], out_hbm.at[pl.ds(obase, B)], osem[p])
        pltpu.async_copy(
            ovb[p].at[1], out_hbm.at[pl.ds(obase + NUM_POINTS, B)], osem[p])

    def owait(p):
        pltpu.make_async_copy(
            ovb[p].at[0], out_hbm.at[pl.ds(0, B)], osem[p]).wait()
        pltpu.make_async_copy(
            ovb[p].at[1], out_hbm.at[pl.ds(0, B)], osem[p]).wait()

    def lvl_body(lvl, carry):
        cbase = lvl * T + sid * TCHUNK
        pltpu.sync_copy(t0_hbm.at[pl.ds(cbase, TCHUNK)],
                        tsh0.at[pl.ds(sid * TCHUNK, TCHUNK)])
        pltpu.sync_copy(t1_hbm.at[pl.ds(cbase, TCHUNK)],
                        tsh1.at[pl.ds(sid * TCHUNK, TCHUNK)])
        plsc.subcore_barrier()

        scale = scv[pl.ds(lvl * LANES, LANES)]
        row0 = 2 * lvl * NUM_POINTS

        make_idx_loop(0, scale, lvl, 0)
        fire(0)
        make_idx_loop(B, scale, lvl, 1)
        gwait(0)
        fire(1)
        make_comb_loop(0)
        ofire(row0, 0, 0)
        make_idx_loop(2 * B, scale, lvl, 0)
        gwait(1)
        fire(0)
        make_comb_loop(1)
        ofire(row0, 1, 1)

        def blk2_body(i, c0):
            b0 = 2 * i
            make_idx_loop((b0 + 1) * B, scale, lvl, 1)
            gwait(0)
            fire(1)
            owait(0)
            make_comb_loop(0)
            ofire(row0, b0, 0)
            make_idx_loop((b0 + 2) * B, scale, lvl, 0)
            gwait(1)
            fire(0)
            owait(1)
            make_comb_loop(1)
            ofire(row0, b0 + 1, 1)
            return c0

        lax.fori_loop(1, NBLK // 2 - 1, blk2_body, 0)

        make_idx_loop(15 * B, scale, lvl, 1)
        gwait(0)
        fire(1)
        owait(0)
        make_comb_loop(0)
        ofire(row0, 14, 0)
        gwait(1)
        owait(1)
        make_comb_loop(1)
        ofire(row0, 15, 1)
        owait(0)
        owait(1)
        plsc.subcore_barrier()
        return carry

    lax.fori_loop(0, L, lvl_body, 0)


@jax.jit
def _encode_sc(xs, ys, zs, t0, t1, scales):
    mesh = plsc.VectorSubcoreMesh(core_axis_name="c", subcore_axis_name="s")
    return pl.kernel(
        _body,
        out_type=jax.ShapeDtypeStruct((L * F * NUM_POINTS,), jnp.float32),
        mesh=mesh,
        scratch_types=[
            pltpu.VMEM((PW,), jnp.float32),
            pltpu.VMEM((PW,), jnp.float32),
            pltpu.VMEM((PW,), jnp.float32),
            pltpu.VMEM((B,), jnp.float32),
            pltpu.VMEM((B,), jnp.float32),
            pltpu.VMEM((B,), jnp.float32),
            pltpu.VMEM((B,), jnp.float32),
            pltpu.VMEM((B,), jnp.float32),
            pltpu.VMEM((B,), jnp.float32),
            pltpu.VMEM((ROWS,), jnp.int32),
            pltpu.VMEM((ROWS,), jnp.int32),
            pltpu.VMEM((ROWS,), jnp.float32),
            pltpu.VMEM((ROWS,), jnp.float32),
            pltpu.VMEM((ROWS,), jnp.float32),
            pltpu.VMEM((ROWS,), jnp.float32),
            pltpu.VMEM((F, B), jnp.float32),
            pltpu.VMEM((F, B), jnp.float32),
            pltpu.VMEM((L * LANES,), jnp.float32),
            pltpu.VMEM_SHARED((T,), jnp.float32),
            pltpu.VMEM_SHARED((T,), jnp.float32),
            pltpu.SemaphoreType.DMA,
            pltpu.SemaphoreType.DMA,
            pltpu.SemaphoreType.DMA,
            pltpu.SemaphoreType.DMA,
            pltpu.SemaphoreType.DMA,
            pltpu.SemaphoreType.DMA,
        ],
    )(xs, ys, zs, t0, t1, scales)


def kernel(x, hash_table):
    xs, ys, zs = x[:, 0], x[:, 1], x[:, 2]
    t0, t1 = hash_table[:, 0], hash_table[:, 1]
    scales = jnp.asarray(
        np.repeat(np.array(SCALES, dtype=np.float32), LANES))
    out = _encode_sc(xs, ys, zs, t0, t1, scales)
    return out.reshape(L * F, NUM_POINTS).T


# speedup vs baseline: 1.1183x; 1.0289x over previous
import functools
import math

import jax
import jax.numpy as jnp
import numpy as np
from jax import lax
from jax.experimental import pallas as pl
from jax.experimental.pallas import tpu as pltpu
from jax.experimental.pallas import tpu_sc as plsc

T = 524288
L = 16
F = 2
N_MIN = 16
N_MAX = 2048
NUM_POINTS = 262144

NC = 2
NS = 16
NW = NC * NS
LANES = 16

PW = NUM_POINTS // NW
B = 512
NBLK = PW // B
BG = B // LANES
ROWS = B * 8
TCHUNK = T // NS
SG = 24
SROWS = SG * 8 * LANES
HROWS = ROWS - SROWS

_GROWTH = math.exp((math.log(N_MAX) - math.log(N_MIN)) / (L - 1))
SCALES = [float(math.floor(N_MIN * (_GROWTH ** l))) for l in range(L)]
P2 = -1640531535
P3 = 805459861
MASK = T - 1


def _body(xs_hbm, ys_hbm, zs_hbm, t0_hbm, t1_hbm, sc_hbm, out_hbm,
          xv, yv, zv, fx0, fy0, fz0, fx1, fy1, fz1,
          idx0, idx1, g0a, g1a, g0b, g1b, ova, ovvb, scv,
          tsh0, tsh1,
          sa0, sa1, sb0, sb1, so0, so1):
    ovb = (ova, ovvb)
    osem = (so0, so1)
    idxb = (idx0, idx1)
    gb = ((g0a, g1a), (g0b, g1b))
    semb = ((sa0, sa1), (sb0, sb1))
    fracb = ((fx0, fy0, fz0), (fx1, fy1, fz1))

    wid = lax.axis_index("s") * NC + lax.axis_index("c")
    sid = lax.axis_index("s")
    wbase = wid * PW
    pltpu.sync_copy(xs_hbm.at[pl.ds(wbase, PW)], xv)
    pltpu.sync_copy(ys_hbm.at[pl.ds(wbase, PW)], yv)
    pltpu.sync_copy(zs_hbm.at[pl.ds(wbase, PW)], zv)
    pltpu.sync_copy(sc_hbm, scv)

    def make_idx_loop(boff, scale, lvl, p):
        idxv = idxb[p]
        fxv, fyv, fzv = fracb[p]

        def hash_groups(g, store):
            sl = pl.ds(g * LANES, LANES)
            xsl = pl.ds(boff + g * LANES, LANES)
            px = xv[xsl] * scale
            py = yv[xsl] * scale
            pz = zv[xsl] * scale
            ix = px.astype(jnp.int32)
            iy = py.astype(jnp.int32)
            iz = pz.astype(jnp.int32)
            fxv[sl] = px - ix.astype(jnp.float32)
            fyv[sl] = py - iy.astype(jnp.float32)
            fzv[sl] = pz - iz.astype(jnp.float32)
            yp = iy * P2
            zp = iz * P3
            hy1 = yp + P2
            hz1 = zp + P3
            a0 = ix & MASK
            a1 = (ix + 1) & MASK
            b00 = (yp ^ zp) & MASK
            b01 = (yp ^ hz1) & MASK
            b10 = (hy1 ^ zp) & MASK
            b11 = (hy1 ^ hz1) & MASK
            store(a0, a1, (b00, b01, b10, b11))

        @plsc.parallel_loop(0, BG, unroll=2)
        def idx_body(g):
            def store(a0, a1, bs):
                rb = g * (8 * LANES)
                for i, a in enumerate((a0, a1)):
                    for j, b in enumerate(bs):
                        idxv[pl.ds(rb + (4 * i + j) * LANES, LANES)] = a ^ b
            hash_groups(g, store)

    def fire(p):
        pltpu.async_copy(tsh0.at[idxb[p]], gb[p][0], semb[p][0])
        pltpu.async_copy(tsh1.at[idxb[p]], gb[p][1], semb[p][1])

    def gwait(p):
        pltpu.make_async_copy(tsh0.at[idxb[p]], gb[p][0], semb[p][0]).wait()
        pltpu.make_async_copy(tsh1.at[idxb[p]], gb[p][1], semb[p][1]).wait()

    def make_comb_loop(p):
        ovl = ovb[p]
        g0v, g1v = gb[p]
        fxv, fyv, fzv = fracb[p]

        @plsc.parallel_loop(0, BG, unroll=2)
        def comb_body(g):
            sl = pl.ds(g * LANES, LANES)
            fx = fxv[sl]
            fy = fyv[sl]
            fz = fzv[sl]
            gx = 1.0 - fx
            gy = 1.0 - fy
            gz = 1.0 - fz
            w00 = gx * gy
            w01 = gx * fy
            w10 = fx * gy
            w11 = fx * fy
            ws = (w00 * gz, w00 * fz, w01 * gz, w01 * fz,
                  w10 * gz, w10 * fz, w11 * gz, w11 * fz)
            rb = g * (8 * LANES)
            acc0 = None
            acc1 = None
            for c in range(8):
                f0 = g0v[pl.ds(rb + c * LANES, LANES)]
                f1 = g1v[pl.ds(rb + c * LANES, LANES)]
                if c == 0:
                    acc0 = ws[c] * f0
                    acc1 = ws[c] * f1
                else:
                    acc0 = acc0 + ws[c] * f0
                    acc1 = acc1 + ws[c] * f1
            ovl[0, sl] = acc0
            ovl[1, sl] = acc1

    def ofire(row0, blk, p):
        obase = row0 + wbase + blk * B
        pltpu.async_copy(ovb[p].at[0], out_hbm.at[pl.ds(obase, B)], osem[p])
        pltpu.async_copy(
            ovb[p].at[1], out_hbm.at[pl.ds(obase + NUM_POINTS, B)], osem[p])

    def owait(p):
        pltpu.make_async_copy(
            ovb[p].at[0], out_hbm.at[pl.ds(0, B)], osem[p]).wait()
        pltpu.make_async_copy(
            ovb[p].at[1], out_hbm.at[pl.ds(0, B)], osem[p]).wait()

    def lvl_body(lvl, carry):
        cbase = lvl * T + sid * TCHUNK
        pltpu.sync_copy(t0_hbm.at[pl.ds(cbase, TCHUNK)],
                        tsh0.at[pl.ds(sid * TCHUNK, TCHUNK)])
        pltpu.sync_copy(t1_hbm.at[pl.ds(cbase, TCHUNK)],
                        tsh1.at[pl.ds(sid * TCHUNK, TCHUNK)])
        plsc.subcore_barrier()

        scale = scv[pl.ds(lvl * LANES, LANES)]
        row0 = 2 * lvl * NUM_POINTS

        make_idx_loop(0, scale, lvl, 0)
        fire(0)
        make_idx_loop(B, scale, lvl, 1)
        fire(1)
        gwait(0)
        make_comb_loop(0)
        ofire(row0, 0, 0)
        make_idx_loop(2 * B, scale, lvl, 0)
        fire(0)
        gwait(1)
        make_comb_loop(1)
        ofire(row0, 1, 1)

        def blk2_body(i, c0):
            b0 = 2 * i
            make_idx_loop((b0 + 1) * B, scale, lvl, 1)
            fire(1)
            gwait(0)
            owait(0)
            make_comb_loop(0)
            ofire(row0, b0, 0)
            make_idx_loop((b0 + 2) * B, scale, lvl, 0)
            fire(0)
            gwait(1)
            owait(1)
            make_comb_loop(1)
            ofire(row0, b0 + 1, 1)
            return c0

        lax.fori_loop(1, NBLK // 2 - 1, blk2_body, 0)

        make_idx_loop(15 * B, scale, lvl, 1)
        fire(1)
        gwait(0)
        owait(0)
        make_comb_loop(0)
        ofire(row0, 14, 0)
        gwait(1)
        owait(1)
        make_comb_loop(1)
        ofire(row0, 15, 1)
        owait(0)
        owait(1)
        plsc.subcore_barrier()
        return carry

    lax.fori_loop(0, L, lvl_body, 0)


@jax.jit
def _encode_sc(xs, ys, zs, t0, t1, scales):
    mesh = plsc.VectorSubcoreMesh(core_axis_name="c", subcore_axis_name="s")
    return pl.kernel(
        _body,
        out_type=jax.ShapeDtypeStruct((L * F * NUM_POINTS,), jnp.float32),
        mesh=mesh,
        scratch_types=[
            pltpu.VMEM((PW,), jnp.float32),
            pltpu.VMEM((PW,), jnp.float32),
            pltpu.VMEM((PW,), jnp.float32),
            pltpu.VMEM((B,), jnp.float32),
            pltpu.VMEM((B,), jnp.float32),
            pltpu.VMEM((B,), jnp.float32),
            pltpu.VMEM((B,), jnp.float32),
            pltpu.VMEM((B,), jnp.float32),
            pltpu.VMEM((B,), jnp.float32),
            pltpu.VMEM((ROWS,), jnp.int32),
            pltpu.VMEM((ROWS,), jnp.int32),
            pltpu.VMEM((ROWS,), jnp.float32),
            pltpu.VMEM((ROWS,), jnp.float32),
            pltpu.VMEM((ROWS,), jnp.float32),
            pltpu.VMEM((ROWS,), jnp.float32),
            pltpu.VMEM((F, B), jnp.float32),
            pltpu.VMEM((F, B), jnp.float32),
            pltpu.VMEM((L * LANES,), jnp.float32),
            pltpu.VMEM_SHARED((T,), jnp.float32),
            pltpu.VMEM_SHARED((T,), jnp.float32),
            pltpu.SemaphoreType.DMA,
            pltpu.SemaphoreType.DMA,
            pltpu.SemaphoreType.DMA,
            pltpu.SemaphoreType.DMA,
            pltpu.SemaphoreType.DMA,
            pltpu.SemaphoreType.DMA,
        ],
    )(xs, ys, zs, t0, t1, scales)


def kernel(x, hash_table):
    xs, ys, zs = x[:, 0], x[:, 1], x[:, 2]
    t0, t1 = hash_table[:, 0], hash_table[:, 1]
    scales = jnp.asarray(
        np.repeat(np.array(SCALES, dtype=np.float32), LANES))
    out = _encode_sc(xs, ys, zs, t0, t1, scales)
    return out.reshape(L * F, NUM_POINTS).T


# speedup vs baseline: 1.1189x; 1.0005x over previous
import math

import jax
import jax.numpy as jnp
import numpy as np
from jax import lax
from jax.experimental import pallas as pl
from jax.experimental.pallas import tpu as pltpu
from jax.experimental.pallas import tpu_sc as plsc

T = 524288
L = 16
F = 2
N_MIN = 16
N_MAX = 2048
NUM_POINTS = 262144

NC = 2
NS = 16
NW = NC * NS
LANES = 16

PW = NUM_POINTS // NW
B = 512
NBLK = PW // B
BG = B // LANES
ROWS = B * 8
TCHUNK = T // NS

_GROWTH = math.exp((math.log(N_MAX) - math.log(N_MIN)) / (L - 1))
SCALES = [float(math.floor(N_MIN * (_GROWTH ** l))) for l in range(L)]
P2 = -1640531535
P3 = 805459861
MASK = T - 1


def _body(xs_hbm, ys_hbm, zs_hbm, t0_hbm, t1_hbm, sc_hbm, out_hbm,
          xv, yv, zv, fx0, fy0, fz0, fx1, fy1, fz1,
          idx0, idx1, g0a, g1a, g0b, g1b, ova, ovvb, scv,
          tsh0, tsh1,
          sa0, sa1, sb0, sb1, so0, so1):
    ovb = (ova, ovvb)
    osem = (so0, so1)
    idxb = (idx0, idx1)
    gb = ((g0a, g1a), (g0b, g1b))
    semb = ((sa0, sa1), (sb0, sb1))
    fracb = ((fx0, fy0, fz0), (fx1, fy1, fz1))

    wid = lax.axis_index("s") * NC + lax.axis_index("c")
    sid = lax.axis_index("s")
    wbase = wid * PW
    pltpu.sync_copy(xs_hbm.at[pl.ds(wbase, PW)], xv)
    pltpu.sync_copy(ys_hbm.at[pl.ds(wbase, PW)], yv)
    pltpu.sync_copy(zs_hbm.at[pl.ds(wbase, PW)], zv)
    pltpu.sync_copy(sc_hbm, scv)

    def make_idx_loop(boff, scale, lvl, p):
        idxv = idxb[p]
        fxv, fyv, fzv = fracb[p]

        def hash_groups(g, store):
            sl = pl.ds(g * LANES, LANES)
            xsl = pl.ds(boff + g * LANES, LANES)
            px = xv[xsl] * scale
            py = yv[xsl] * scale
            pz = zv[xsl] * scale
            ix = px.astype(jnp.int32)
            iy = py.astype(jnp.int32)
            iz = pz.astype(jnp.int32)
            fxv[sl] = px - ix.astype(jnp.float32)
            fyv[sl] = py - iy.astype(jnp.float32)
            fzv[sl] = pz - iz.astype(jnp.float32)
            yp = iy * P2
            zp = iz * P3
            hy1 = yp + P2
            hz1 = zp + P3
            a0 = ix & MASK
            a1 = (ix + 1) & MASK
            b00 = (yp ^ zp) & MASK
            b01 = (yp ^ hz1) & MASK
            b10 = (hy1 ^ zp) & MASK
            b11 = (hy1 ^ hz1) & MASK
            store(a0, a1, (b00, b01, b10, b11))

        @plsc.parallel_loop(0, BG, unroll=2)
        def idx_body(g):
            def store(a0, a1, bs):
                rb = g * (8 * LANES)
                for i, a in enumerate((a0, a1)):
                    for j, b in enumerate(bs):
                        idxv[pl.ds(rb + (4 * i + j) * LANES, LANES)] = a ^ b
            hash_groups(g, store)

    def fire(p):
        pltpu.async_copy(tsh0.at[idxb[p]], gb[p][0], semb[p][0])
        pltpu.async_copy(tsh1.at[idxb[p]], gb[p][1], semb[p][1])

    def gwait(p):
        pltpu.make_async_copy(tsh0.at[idxb[p]], gb[p][0], semb[p][0]).wait()
        pltpu.make_async_copy(tsh1.at[idxb[p]], gb[p][1], semb[p][1]).wait()

    def make_comb_loop(p):
        ovl = ovb[p]
        g0v, g1v = gb[p]
        fxv, fyv, fzv = fracb[p]

        @plsc.parallel_loop(0, BG, unroll=2)
        def comb_body(g):
            sl = pl.ds(g * LANES, LANES)
            fx = fxv[sl]
            fy = fyv[sl]
            fz = fzv[sl]
            gx = 1.0 - fx
            gy = 1.0 - fy
            gz = 1.0 - fz
            w00 = gx * gy
            w01 = gx * fy
            w10 = fx * gy
            w11 = fx * fy
            ws = (w00 * gz, w00 * fz, w01 * gz, w01 * fz,
                  w10 * gz, w10 * fz, w11 * gz, w11 * fz)
            rb = g * (8 * LANES)
            acc0 = None
            acc1 = None
            for c in range(8):
                f0 = g0v[pl.ds(rb + c * LANES, LANES)]
                f1 = g1v[pl.ds(rb + c * LANES, LANES)]
                if c == 0:
                    acc0 = ws[c] * f0
                    acc1 = ws[c] * f1
                else:
                    acc0 = acc0 + ws[c] * f0
                    acc1 = acc1 + ws[c] * f1
            ovl[0, sl] = acc0
            ovl[1, sl] = acc1

    def ofire(row0, blk, p):
        obase = row0 + wbase + blk * B
        pltpu.async_copy(ovb[p].at[0], out_hbm.at[pl.ds(obase, B)], osem[p])
        pltpu.async_copy(
            ovb[p].at[1], out_hbm.at[pl.ds(obase + NUM_POINTS, B)], osem[p])

    def owait(p):
        pltpu.make_async_copy(
            ovb[p].at[0], out_hbm.at[pl.ds(0, B)], osem[p]).wait()
        pltpu.make_async_copy(
            ovb[p].at[1], out_hbm.at[pl.ds(0, B)], osem[p]).wait()

    def lvl_body(lvl, carry):
        cbase = lvl * T + sid * TCHUNK
        pltpu.sync_copy(t0_hbm.at[pl.ds(cbase, TCHUNK)],
                        tsh0.at[pl.ds(sid * TCHUNK, TCHUNK)])
        pltpu.sync_copy(t1_hbm.at[pl.ds(cbase, TCHUNK)],
                        tsh1.at[pl.ds(sid * TCHUNK, TCHUNK)])
        plsc.subcore_barrier()

        scale = scv[pl.ds(lvl * LANES, LANES)]
        row0 = 2 * lvl * NUM_POINTS

        make_idx_loop(0, scale, lvl, 0)
        fire(0)
        make_idx_loop(B, scale, lvl, 1)
        fire(1)
        gwait(0)
        make_comb_loop(0)
        ofire(row0, 0, 0)
        make_idx_loop(2 * B, scale, lvl, 0)
        fire(0)
        gwait(1)
        make_comb_loop(1)
        ofire(row0, 1, 1)

        def blk2_body(i, c0):
            b0 = 2 * i
            make_idx_loop((b0 + 1) * B, scale, lvl, 1)
            fire(1)
            gwait(0)
            owait(0)
            make_comb_loop(0)
            ofire(row0, b0, 0)
            make_idx_loop((b0 + 2) * B, scale, lvl, 0)
            fire(0)
            gwait(1)
            owait(1)
            make_comb_loop(1)
            ofire(row0, b0 + 1, 1)
            return c0

        lax.fori_loop(1, NBLK // 2 - 1, blk2_body, 0)

        make_idx_loop(15 * B, scale, lvl, 1)
        fire(1)
        gwait(0)
        owait(0)
        make_comb_loop(0)
        ofire(row0, 14, 0)
        gwait(1)
        owait(1)
        make_comb_loop(1)
        ofire(row0, 15, 1)
        owait(0)
        owait(1)
        plsc.subcore_barrier()
        return carry

    lax.fori_loop(0, L, lvl_body, 0)


@jax.jit
def _encode_sc(xs, ys, zs, t0, t1, scales):
    mesh = plsc.VectorSubcoreMesh(core_axis_name="c", subcore_axis_name="s")
    return pl.kernel(
        _body,
        out_type=jax.ShapeDtypeStruct((L * F * NUM_POINTS,), jnp.float32),
        mesh=mesh,
        scratch_types=[
            pltpu.VMEM((PW,), jnp.float32),
            pltpu.VMEM((PW,), jnp.float32),
            pltpu.VMEM((PW,), jnp.float32),
            pltpu.VMEM((B,), jnp.float32),
            pltpu.VMEM((B,), jnp.float32),
            pltpu.VMEM((B,), jnp.float32),
            pltpu.VMEM((B,), jnp.float32),
            pltpu.VMEM((B,), jnp.float32),
            pltpu.VMEM((B,), jnp.float32),
            pltpu.VMEM((ROWS,), jnp.int32),
            pltpu.VMEM((ROWS,), jnp.int32),
            pltpu.VMEM((ROWS,), jnp.float32),
            pltpu.VMEM((ROWS,), jnp.float32),
            pltpu.VMEM((ROWS,), jnp.float32),
            pltpu.VMEM((ROWS,), jnp.float32),
            pltpu.VMEM((F, B), jnp.float32),
            pltpu.VMEM((F, B), jnp.float32),
            pltpu.VMEM((L * LANES,), jnp.float32),
            pltpu.VMEM_SHARED((T,), jnp.float32),
            pltpu.VMEM_SHARED((T,), jnp.float32),
            pltpu.SemaphoreType.DMA,
            pltpu.SemaphoreType.DMA,
            pltpu.SemaphoreType.DMA,
            pltpu.SemaphoreType.DMA,
            pltpu.SemaphoreType.DMA,
            pltpu.SemaphoreType.DMA,
        ],
    )(xs, ys, zs, t0, t1, scales)


def kernel(x, hash_table):
    xs, ys, zs = x[:, 0], x[:, 1], x[:, 2]
    t0, t1 = hash_table[:, 0], hash_table[:, 1]
    scales = jnp.asarray(
        np.repeat(np.array(SCALES, dtype=np.float32), LANES))
    out = _encode_sc(xs, ys, zs, t0, t1, scales)
    return out.reshape(L * F, NUM_POINTS).T
